# baseline (speedup 1.0000x reference)
import jax
import jax.numpy as jnp
from jax import lax
from jax.experimental import pallas as pl

_TOTAL = 32 * 3 * 16 * 128 * 128
_W = 128
_ROWS = _TOTAL // _W
_GRID = 16
_BLK = _ROWS // _GRID
_TR = 256
_NT = _BLK // _TR


def _kl_block(p1_ref, p2_ref, out_ref):
    def body(i, acc):
        r = i * _TR
        p1 = p1_ref[pl.ds(r, _TR), :]
        p2 = p2_ref[pl.ds(r, _TR), :]
        np1 = 1.0 - p1
        np2 = 1.0 - p2
        kl = p1 * jnp.log(p1 / p2) + np1 * jnp.log(np1 / np2)
        return acc + kl

    acc = lax.fori_loop(
        0, _NT, body, jnp.zeros((_TR, _W), jnp.float32), unroll=4
    )
    s = jnp.sum(acc).reshape(1, 1)

    @pl.when(pl.program_id(0) == 0)
    def _init():
        out_ref[...] = s

    @pl.when(pl.program_id(0) != 0)
    def _acc():
        out_ref[...] += s


def kernel(pred1, pred2):
    p1 = pred1.reshape(_ROWS, _W)
    p2 = pred2.reshape(_ROWS, _W)
    out = pl.pallas_call(
        _kl_block,
        grid=(_GRID,),
        in_specs=[
            pl.BlockSpec((_BLK, _W), lambda i: (i, 0)),
            pl.BlockSpec((_BLK, _W), lambda i: (i, 0)),
        ],
        out_specs=pl.BlockSpec((1, 1), lambda i: (0, 0)),
        out_shape=jax.ShapeDtypeStruct((1, 1), jnp.float32),
    )(p1, p2)
    return out[0, 0]


# speedup vs baseline: 1.1541x; 1.1541x over previous
import jax
import jax.numpy as jnp
from jax import lax
from jax.experimental import pallas as pl

_TOTAL = 32 * 3 * 16 * 128 * 128
_W = 128
_ROWS = _TOTAL // _W
_GRID = 16
_BLK = _ROWS // _GRID
_TR = 64
_NT = _BLK // _TR


def _kl_block(p1_ref, p2_ref, out_ref):
    def body(i, acc):
        r = i * _TR
        p1 = p1_ref[pl.ds(r, _TR), :]
        p2 = p2_ref[pl.ds(r, _TR), :]
        np1 = 1.0 - p1
        np2 = 1.0 - p2
        kl = p1 * jnp.log(p1 / p2) + np1 * jnp.log(np1 / np2)
        return acc + kl

    acc = lax.fori_loop(
        0, _NT, body, jnp.zeros((_TR, _W), jnp.float32), unroll=16
    )
    s = jnp.sum(acc).reshape(1, 1)

    @pl.when(pl.program_id(0) == 0)
    def _init():
        out_ref[...] = s

    @pl.when(pl.program_id(0) != 0)
    def _acc():
        out_ref[...] += s


def kernel(pred1, pred2):
    p1 = pred1.reshape(_ROWS, _W)
    p2 = pred2.reshape(_ROWS, _W)
    out = pl.pallas_call(
        _kl_block,
        grid=(_GRID,),
        in_specs=[
            pl.BlockSpec((_BLK, _W), lambda i: (i, 0)),
            pl.BlockSpec((_BLK, _W), lambda i: (i, 0)),
        ],
        out_specs=pl.BlockSpec((1, 1), lambda i: (0, 0)),
        out_shape=jax.ShapeDtypeStruct((1, 1), jnp.float32),
    )(p1, p2)
    return out[0, 0]
